# baseline (speedup 1.0000x reference)
import functools

import jax
import jax.numpy as jnp
from jax import lax
from jax.experimental import pallas as pl
from jax.experimental.pallas import tpu as pltpu
from jax.experimental.pallas import tpu_sc as plsc


def _make_sc_broadcast(batch, n_sents, emb, dtype):
    info = plsc.get_sparse_core_info()
    nc, ns = info.num_cores, info.num_subcores
    nw = nc * ns
    b_per_w = batch // nw
    mesh = plsc.VectorSubcoreMesh(core_axis_name="c", subcore_axis_name="s")

    half = b_per_w // 2

    @functools.partial(
        pl.kernel,
        mesh=mesh,
        out_type=jax.ShapeDtypeStruct((batch, n_sents, emb), dtype),
        scratch_types=[
            pltpu.VMEM((n_sents, emb), dtype),
            pltpu.VMEM_SHARED((n_sents, emb), dtype),
            pltpu.SemaphoreType.DMA,
        ],
    )
    def k(tbl_hbm, out_hbm, tbl_v, tbl_sh, sem):
        sid = lax.axis_index("s")
        wid = sid * nc + lax.axis_index("c")
        base = wid * b_per_w
        pltpu.sync_copy(tbl_hbm.at[pl.ds(0, n_sents)], tbl_v)
        copies = [
            pltpu.async_copy(tbl_v, out_hbm.at[base + i], sem)
            for i in range(half)
        ]

        @pl.when(sid == 0)
        def _fill_shared():
            pltpu.sync_copy(tbl_hbm.at[pl.ds(0, n_sents)], tbl_sh)

        plsc.subcore_barrier()
        copies += [
            pltpu.async_copy(tbl_sh, out_hbm.at[base + half + i], sem)
            for i in range(b_per_w - half)
        ]
        for c in copies:
            c.wait()

    return k


def _tc_bcast_body(tbl_ref, out_ref):
    n_sents = out_ref.shape[1]
    out_ref[...] = jnp.broadcast_to(tbl_ref[:n_sents][None, :, :], out_ref.shape)


def _tc_broadcast(pos_table, batch, n_sents, emb, b_blk):
    return pl.pallas_call(
        _tc_bcast_body,
        grid=(batch // b_blk,),
        in_specs=[pl.BlockSpec(pos_table.shape, lambda i: (0, 0))],
        out_specs=pl.BlockSpec((b_blk, n_sents, emb), lambda i: (i, 0, 0)),
        out_shape=jax.ShapeDtypeStruct((batch, n_sents, emb), pos_table.dtype),
    )(pos_table)


def kernel(top_vecs, tok_struct_vec, sent_struct_vec, pos_table):
    batch, n_sents = top_vecs.shape[0], top_vecs.shape[1]
    emb = pos_table.shape[1]
    sc_fn = _make_sc_broadcast(batch, n_sents, emb, pos_table.dtype)
    return sc_fn(pos_table)


# speedup vs baseline: 1.0534x; 1.0534x over previous
import functools

import jax
import jax.numpy as jnp
from jax import lax
from jax.experimental import pallas as pl
from jax.experimental.pallas import tpu as pltpu
from jax.experimental.pallas import tpu_sc as plsc


def _make_sc_broadcast(batch, n_sents, emb, dtype):
    info = plsc.get_sparse_core_info()
    nc, ns = info.num_cores, info.num_subcores
    nw = nc * ns
    b_per_w = batch // nw
    mesh = plsc.VectorSubcoreMesh(core_axis_name="c", subcore_axis_name="s")

    half = b_per_w // 2

    @functools.partial(
        pl.kernel,
        mesh=mesh,
        out_type=jax.ShapeDtypeStruct((batch, n_sents, emb), dtype),
        scratch_types=[
            pltpu.VMEM((n_sents, emb), dtype),
            pltpu.VMEM_SHARED((n_sents, emb), dtype),
            pltpu.SemaphoreType.DMA,
        ],
    )
    def k(tbl_hbm, out_hbm, tbl_v, tbl_sh, sem):
        sid = lax.axis_index("s")
        wid = sid * nc + lax.axis_index("c")
        base = wid * b_per_w
        pltpu.sync_copy(tbl_hbm.at[pl.ds(0, n_sents)], tbl_v)

        @pl.when(sid == 0)
        def _fill_shared():
            pltpu.sync_copy(tbl_v, tbl_sh)

        plsc.subcore_barrier()
        copies = []
        for i in range(half):
            copies.append(pltpu.async_copy(tbl_v, out_hbm.at[base + i], sem))
            copies.append(
                pltpu.async_copy(tbl_sh, out_hbm.at[base + half + i], sem)
            )
        for c in copies:
            c.wait()

    return k


def _tc_bcast_body(tbl_ref, out_ref):
    n_sents = out_ref.shape[1]
    out_ref[...] = jnp.broadcast_to(tbl_ref[:n_sents][None, :, :], out_ref.shape)


def _tc_broadcast(pos_table, batch, n_sents, emb, b_blk):
    return pl.pallas_call(
        _tc_bcast_body,
        grid=(batch // b_blk,),
        in_specs=[pl.BlockSpec(pos_table.shape, lambda i: (0, 0))],
        out_specs=pl.BlockSpec((b_blk, n_sents, emb), lambda i: (i, 0, 0)),
        out_shape=jax.ShapeDtypeStruct((batch, n_sents, emb), pos_table.dtype),
    )(pos_table)


def kernel(top_vecs, tok_struct_vec, sent_struct_vec, pos_table):
    batch, n_sents = top_vecs.shape[0], top_vecs.shape[1]
    emb = pos_table.shape[1]
    sc_fn = _make_sc_broadcast(batch, n_sents, emb, pos_table.dtype)
    return sc_fn(pos_table)


# speedup vs baseline: 1.0563x; 1.0027x over previous
import functools

import jax
from jax import lax
from jax.experimental import pallas as pl
from jax.experimental.pallas import tpu as pltpu
from jax.experimental.pallas import tpu_sc as plsc


def _make_sc_broadcast(batch, n_sents, emb, dtype):
    info = plsc.get_sparse_core_info()
    nc, ns = info.num_cores, info.num_subcores
    nw = nc * ns
    b_per_w = batch // nw
    mesh = plsc.VectorSubcoreMesh(core_axis_name="c", subcore_axis_name="s")

    half = b_per_w // 2

    @functools.partial(
        pl.kernel,
        mesh=mesh,
        out_type=jax.ShapeDtypeStruct((batch, n_sents, emb), dtype),
        scratch_types=[
            pltpu.VMEM((n_sents, emb), dtype),
            pltpu.VMEM_SHARED((n_sents, emb), dtype),
            pltpu.SemaphoreType.DMA,
        ],
    )
    def k(tbl_hbm, out_hbm, tbl_v, tbl_sh, sem):
        sid = lax.axis_index("s")
        wid = sid * nc + lax.axis_index("c")
        base = wid * b_per_w
        pltpu.sync_copy(tbl_hbm.at[pl.ds(0, n_sents)], tbl_v)

        @pl.when(sid == 0)
        def _fill_shared():
            pltpu.sync_copy(tbl_v, tbl_sh)

        plsc.subcore_barrier()
        copies = []
        for i in range(half):
            copies.append(pltpu.async_copy(tbl_v, out_hbm.at[base + i], sem))
            copies.append(
                pltpu.async_copy(tbl_sh, out_hbm.at[base + half + i], sem)
            )
        for i in range(2 * half, b_per_w):
            copies.append(pltpu.async_copy(tbl_v, out_hbm.at[base + i], sem))
        for c in copies:
            c.wait()

    return k


def kernel(top_vecs, tok_struct_vec, sent_struct_vec, pos_table):
    batch, n_sents = top_vecs.shape[0], top_vecs.shape[1]
    emb = pos_table.shape[1]
    sc_fn = _make_sc_broadcast(batch, n_sents, emb, pos_table.dtype)
    return sc_fn(pos_table)
